# Initial kernel scaffold; baseline (speedup 1.0000x reference)
import functools

import jax
import jax.numpy as jnp
from jax import lax
from jax.experimental import pallas as pl
from jax.experimental.pallas import tpu as pltpu
from jax.experimental.pallas import tpu_sc as plsc

N = 32768
D = 128
NSEG = 64
L = 16
NC, NS = 2, 16
NW = NC * NS
ROWS_W = N // NW
R = 256
NCH = ROWS_W // R
GPC = R // L
CPD = D // L
EPS = 1e-06
LAMBDA_PCSL = 0.1
LAMBDA_GPAL = 0.1


def _sc_partials(features, patient_ids):
    mesh = plsc.VectorSubcoreMesh(core_axis_name="c", subcore_axis_name="s")

    @functools.partial(
        pl.kernel,
        out_type=[
            jax.ShapeDtypeStruct((NW, NSEG, D), jnp.float32),
            jax.ShapeDtypeStruct((NW, L), jnp.float32),
        ],
        mesh=mesh,
        scratch_types=[
            pltpu.VMEM((ROWS_W,), jnp.int32),
            pltpu.VMEM((R, D), jnp.float32),
            pltpu.VMEM((NSEG, D), jnp.float32),
            pltpu.VMEM((CPD, L), jnp.float32),
            pltpu.VMEM((L,), jnp.float32),
        ],
    )
    def k(feat_hbm, ids_hbm, psum_hbm, psq_hbm, ids_v, buf_v, acc_v, sq_v,
          sqout_v):
        wid = lax.axis_index("s") * NC + lax.axis_index("c")
        base = wid * ROWS_W
        pltpu.sync_copy(ids_hbm.at[pl.ds(base, ROWS_W)], ids_v)

        zeros = jnp.zeros((L,), jnp.float32)

        def zacc(i, _):
            for c in range(CPD):
                acc_v[i, pl.ds(c * L, L)] = zeros
            return 0

        lax.fori_loop(0, NSEG, zacc, 0)
        for c in range(CPD):
            sq_v[c, :] = zeros

        iota = lax.iota(jnp.int32, L)
        col_idx = [iota + c * L for c in range(CPD)]

        for ch in range(NCH):
            pltpu.sync_copy(feat_hbm.at[pl.ds(base + ch * R, R), :], buf_v)

            def grp(g, _):
                v = ids_v[pl.ds(ch * R + g * L, L)]
                for r in range(L):
                    seg = jnp.take(v, jnp.full((L,), r, jnp.int32),
                                   mode="promise_in_bounds")
                    row = g * L + r
                    for c in range(CPD):
                        data = buf_v[row, pl.ds(c * L, L)]
                        plsc.addupdate_scatter(acc_v, [seg, col_idx[c]], data)
                        plsc.addupdate(sq_v.at[c], data * data)
                return 0

            lax.fori_loop(0, GPC, grp, 0)

        tot = sq_v[0, :]
        for c in range(1, CPD):
            tot = tot + sq_v[c, :]
        sqout_v[...] = tot
        pltpu.sync_copy(acc_v, psum_hbm.at[wid])
        pltpu.sync_copy(sqout_v, psq_hbm.at[wid])

    return k(features, patient_ids)


def _epi_body(ps_ref, sq_ref, ids_ref, out_ref):
    sums = ps_ref[0]
    for t in range(1, NW):
        sums = sums + ps_ref[t]
    total_sq = jnp.sum(sq_ref[...])

    seg_iota = lax.broadcasted_iota(jnp.int32, (NSEG, D), 0)

    def cnt_body(r, cm):
        row = ids_ref[pl.ds(r, 1), :]
        m = jnp.broadcast_to(row, (NSEG, D)) == seg_iota
        return cm + m.astype(jnp.float32)

    cnt = lax.fori_loop(0, N // D, cnt_body,
                        jnp.zeros((NSEG, D), jnp.float32))

    safe = jnp.maximum(cnt, 1.0)
    cent = sums / safe
    csq = cent * cent
    within = total_sq - jnp.sum(cnt * csq)
    validf = (cnt > 0).astype(jnp.float32)
    kseg = jnp.sum(validf) / D
    csqsum = jnp.sum(validf * csq)
    svec = jnp.sum(validf * cent, axis=0, keepdims=True)
    ssq = jnp.sum(svec * svec)
    between = kseg * csqsum - ssq
    loss_pcsl = within / (between + EPS)
    loss_gpal = (csqsum - ssq / kseg) / kseg
    out_ref[0, 0] = LAMBDA_PCSL * loss_pcsl + LAMBDA_GPAL * loss_gpal


def kernel(features, patient_ids):
    psum, psq = _sc_partials(features, patient_ids)
    ids2d = patient_ids.reshape(N // D, D)
    out = pl.pallas_call(
        _epi_body,
        out_shape=jax.ShapeDtypeStruct((1, 1), jnp.float32),
    )(psum, psq, ids2d)
    return out[0, 0]


# speedup vs baseline: 3.7778x; 3.7778x over previous
import functools

import jax
import jax.numpy as jnp
from jax import lax
from jax.experimental import pallas as pl
from jax.experimental.pallas import tpu as pltpu
from jax.experimental.pallas import tpu_sc as plsc

N = 32768
D = 128
NSEG = 64
L = 16
NC, NS = 2, 16
NW = NC * NS
ROWS_W = N // NW
R = 256
NCH = ROWS_W // R
GPC = R // L
CPD = D // L
EPS = 1e-06
LAMBDA_PCSL = 0.1
LAMBDA_GPAL = 0.1


_GDN = lax.GatherDimensionNumbers(
    offset_dims=(), collapsed_slice_dims=(0,), start_index_map=(0,))


def _splat_lane(v, r):
    idx = jnp.full((L, 1), r, jnp.int32)
    return lax.gather(v, idx, _GDN, (1,),
                      mode=lax.GatherScatterMode.PROMISE_IN_BOUNDS)


def _sc_partials(features, patient_ids):
    mesh = plsc.VectorSubcoreMesh(core_axis_name="c", subcore_axis_name="s")

    @functools.partial(
        pl.kernel,
        out_type=[
            jax.ShapeDtypeStruct((NW, NSEG * D), jnp.float32),
            jax.ShapeDtypeStruct((NW, L), jnp.float32),
        ],
        mesh=mesh,
        compiler_params=pltpu.CompilerParams(needs_layout_passes=False),
        scratch_types=[
            pltpu.VMEM((ROWS_W,), jnp.int32),
            pltpu.VMEM((R, D), jnp.float32),
            pltpu.VMEM((NSEG * D,), jnp.float32),
            pltpu.VMEM((CPD, L), jnp.float32),
            pltpu.VMEM((L,), jnp.float32),
        ],
    )
    def k(feat_hbm, ids_hbm, psum_hbm, psq_hbm, ids_v, buf_v, acc_v, sq_v,
          sqout_v):
        wid = lax.axis_index("s") * NC + lax.axis_index("c")
        base = wid * ROWS_W
        pltpu.sync_copy(ids_hbm.at[pl.ds(base, ROWS_W)], ids_v)

        zeros = jnp.zeros((L,), jnp.float32)

        def zacc(i, _):
            acc_v[pl.ds(i * L, L)] = zeros
            return 0

        lax.fori_loop(0, NSEG * D // L, zacc, 0)
        for c in range(CPD):
            sq_v[c, :] = zeros

        iota = lax.iota(jnp.int32, L)
        col_idx = [iota + c * L for c in range(CPD)]

        for ch in range(NCH):
            pltpu.sync_copy(feat_hbm.at[pl.ds(base + ch * R, R), :], buf_v)

            def grp(g, _):
                v = ids_v[pl.ds(ch * R + g * L, L)]
                for r in range(L):
                    seg = _splat_lane(v, r)
                    seg_base = seg * D
                    row = g * L + r
                    for c in range(CPD):
                        data = buf_v[row, pl.ds(c * L, L)]
                        plsc.addupdate_scatter(
                            acc_v, [seg_base + col_idx[c]], data)
                        plsc.addupdate(sq_v.at[c], data * data)
                return 0

            lax.fori_loop(0, GPC, grp, 0)

        tot = sq_v[0, :]
        for c in range(1, CPD):
            tot = tot + sq_v[c, :]
        sqout_v[...] = tot
        pltpu.sync_copy(acc_v, psum_hbm.at[wid])
        pltpu.sync_copy(sqout_v, psq_hbm.at[wid])

    return k(features, patient_ids)


def _epi_body(ps_ref, sq_ref, ids_ref, out_ref):
    sums = ps_ref[0]
    for t in range(1, NW):
        sums = sums + ps_ref[t]
    total_sq = jnp.sum(sq_ref[...])

    seg_iota = lax.broadcasted_iota(jnp.int32, (NSEG, D), 0)

    def cnt_body(r, cm):
        row = ids_ref[pl.ds(r, 1), :]
        m = jnp.broadcast_to(row, (NSEG, D)) == seg_iota
        return cm + m.astype(jnp.float32)

    cnt = lax.fori_loop(0, N // D, cnt_body,
                        jnp.zeros((NSEG, D), jnp.float32))
    cnt = jnp.broadcast_to(jnp.sum(cnt, axis=1, keepdims=True), (NSEG, D))

    safe = jnp.maximum(cnt, 1.0)
    cent = sums / safe
    csq = cent * cent
    within = total_sq - jnp.sum(cnt * csq)
    validf = (cnt > 0).astype(jnp.float32)
    kseg = jnp.sum(validf) / D
    csqsum = jnp.sum(validf * csq)
    svec = jnp.sum(validf * cent, axis=0, keepdims=True)
    ssq = jnp.sum(svec * svec)
    between = kseg * csqsum - ssq
    loss_pcsl = within / (between + EPS)
    loss_gpal = (csqsum - ssq / kseg) / kseg
    loss = LAMBDA_PCSL * loss_pcsl + LAMBDA_GPAL * loss_gpal
    out_ref[...] = jnp.broadcast_to(loss, (1, 1))


def kernel(features, patient_ids):
    psum, psq = _sc_partials(features, patient_ids)
    ids2d = patient_ids.reshape(N // D, D)
    out = pl.pallas_call(
        _epi_body,
        out_shape=jax.ShapeDtypeStruct((1, 1), jnp.float32),
    )(psum.reshape(NW, NSEG, D), psq, ids2d)
    return out[0, 0]


# speedup vs baseline: 6.4596x; 1.7099x over previous
import functools

import jax
import jax.numpy as jnp
from jax import lax
from jax.experimental import pallas as pl
from jax.experimental.pallas import tpu as pltpu
from jax.experimental.pallas import tpu_sc as plsc

N = 32768
D = 128
NSEG = 64
L = 16
NC, NS = 2, 16
NW = NC * NS
ROWS_W = N // NW
R = 256
NCH = ROWS_W // R
GPC = R // L
CPD = D // L
EPS = 1e-06
LAMBDA_PCSL = 0.1
LAMBDA_GPAL = 0.1


def _tree_sum(vs):
    while len(vs) > 1:
        vs = [a + b for a, b in zip(vs[::2], vs[1::2])]
    return vs[0]


def _sc_partials(features, patient_ids):
    mesh = plsc.VectorSubcoreMesh(core_axis_name="c", subcore_axis_name="s")

    @functools.partial(
        pl.kernel,
        out_type=[
            jax.ShapeDtypeStruct((NW, NSEG * D), jnp.float32),
            jax.ShapeDtypeStruct((NW, L), jnp.float32),
        ],
        mesh=mesh,
        compiler_params=pltpu.CompilerParams(needs_layout_passes=False),
        scratch_types=[
            pltpu.VMEM((ROWS_W,), jnp.int32),
            pltpu.VMEM((R, D), jnp.float32),
            pltpu.VMEM((R, D), jnp.float32),
            pltpu.VMEM((NSEG * D,), jnp.float32),
            pltpu.VMEM((L,), jnp.float32),
            pltpu.SemaphoreType.DMA,
            pltpu.SemaphoreType.DMA,
        ],
    )
    def k(feat_hbm, ids_hbm, psum_hbm, psq_hbm, ids_v, buf0_v, buf1_v,
          acc_v, sqout_v, sem0, sem1):
        wid = lax.axis_index("s") * NC + lax.axis_index("c")
        base = wid * ROWS_W
        bufs = [buf0_v, buf1_v]
        sems = [sem0, sem1]

        handles = {}
        for ch in range(min(2, NCH)):
            handles[ch] = pltpu.async_copy(
                feat_hbm.at[pl.ds(base + ch * R, R), :], bufs[ch % 2],
                sems[ch % 2])
        pltpu.sync_copy(ids_hbm.at[pl.ds(base, ROWS_W)], ids_v)

        zeros = jnp.zeros((L,), jnp.float32)

        def zacc(i, _):
            acc_v[pl.ds(i * L, L)] = zeros
            return 0

        lax.fori_loop(0, NSEG * D // L, zacc, 0)

        iota = lax.iota(jnp.int32, L)
        sqs = tuple(zeros for _ in range(CPD))

        for ch in range(NCH):
            buf_v = bufs[ch % 2]
            handles[ch].wait()

            def grp(g, sqs):
                rb = g * L
                v = ids_v[pl.ds(ch * R + g * L, L)]
                lo = jnp.min(v)
                hi = jnp.max(v)

                def fast(sqs):
                    new = []
                    sb = lo * D
                    for c in range(CPD):
                        data = [buf_v[rb + r, pl.ds(c * L, L)]
                                for r in range(L)]
                        s = _tree_sum(data)
                        sq = _tree_sum([x * x for x in data])
                        plsc.addupdate(acc_v.at[pl.ds(sb + c * L, L)], s)
                        new.append(sqs[c] + sq)
                    return tuple(new)

                def slow(sqs):
                    sqs = list(sqs)
                    for r in range(L):
                        seg = jnp.sum(jnp.where(iota == r, v, 0))
                        sb = seg * D
                        for c in range(CPD):
                            data = buf_v[rb + r, pl.ds(c * L, L)]
                            plsc.addupdate(
                                acc_v.at[pl.ds(sb + c * L, L)], data)
                            sqs[c] = sqs[c] + data * data
                    return tuple(sqs)

                return lax.cond(lo == hi, fast, slow, sqs)

            sqs = lax.fori_loop(0, GPC, grp, sqs)
            if ch + 2 < NCH:
                handles[ch + 2] = pltpu.async_copy(
                    feat_hbm.at[pl.ds(base + (ch + 2) * R, R), :], buf_v,
                    sems[ch % 2])

        sqout_v[...] = _tree_sum(list(sqs))
        pltpu.sync_copy(acc_v, psum_hbm.at[wid])
        pltpu.sync_copy(sqout_v, psq_hbm.at[wid])

    return k(features, patient_ids)


def _epi_body(ps_ref, sq_ref, ids_ref, out_ref):
    sums = ps_ref[0]
    for t in range(1, NW):
        sums = sums + ps_ref[t]
    total_sq = jnp.sum(sq_ref[...])

    seg_iota = lax.broadcasted_iota(jnp.int32, (NSEG, D), 0)

    def cnt_body(r, cm):
        row = ids_ref[pl.ds(r, 1), :]
        m = jnp.broadcast_to(row, (NSEG, D)) == seg_iota
        return cm + m.astype(jnp.float32)

    cnt = lax.fori_loop(0, N // D, cnt_body,
                        jnp.zeros((NSEG, D), jnp.float32))
    cnt = jnp.broadcast_to(jnp.sum(cnt, axis=1, keepdims=True), (NSEG, D))

    safe = jnp.maximum(cnt, 1.0)
    cent = sums / safe
    csq = cent * cent
    within = total_sq - jnp.sum(cnt * csq)
    validf = (cnt > 0).astype(jnp.float32)
    kseg = jnp.sum(validf) / D
    csqsum = jnp.sum(validf * csq)
    svec = jnp.sum(validf * cent, axis=0, keepdims=True)
    ssq = jnp.sum(svec * svec)
    between = kseg * csqsum - ssq
    loss_pcsl = within / (between + EPS)
    loss_gpal = (csqsum - ssq / kseg) / kseg
    loss = LAMBDA_PCSL * loss_pcsl + LAMBDA_GPAL * loss_gpal
    out_ref[...] = jnp.broadcast_to(loss, (1, 1))


def kernel(features, patient_ids):
    psum, psq = _sc_partials(features, patient_ids)
    ids2d = patient_ids.reshape(N // D, D)
    out = pl.pallas_call(
        _epi_body,
        out_shape=jax.ShapeDtypeStruct((1, 1), jnp.float32),
    )(psum.reshape(NW, NSEG, D), psq, ids2d)
    return out[0, 0]


# speedup vs baseline: 6.8564x; 1.0614x over previous
import functools

import jax
import jax.numpy as jnp
from jax import lax
from jax.experimental import pallas as pl
from jax.experimental.pallas import tpu as pltpu
from jax.experimental.pallas import tpu_sc as plsc

N = 32768
D = 128
NSEG = 64
L = 16
NC, NS = 2, 16
NW = NC * NS
ROWS_W = N // NW
R = 256
NCH = ROWS_W // R
GPC = R // L
CPD = D // L
EPS = 1e-06
LAMBDA_PCSL = 0.1
LAMBDA_GPAL = 0.1


def _tree_sum(vs):
    while len(vs) > 1:
        vs = [a + b for a, b in zip(vs[::2], vs[1::2])]
    return vs[0]


def _sc_partials(features, patient_ids):
    mesh = plsc.VectorSubcoreMesh(core_axis_name="c", subcore_axis_name="s")

    @functools.partial(
        pl.kernel,
        out_type=[
            jax.ShapeDtypeStruct((NW, NSEG, D), jnp.float32),
            jax.ShapeDtypeStruct((NW, L), jnp.float32),
        ],
        mesh=mesh,
        compiler_params=pltpu.CompilerParams(needs_layout_passes=False),
        scratch_types=[
            pltpu.VMEM((ROWS_W,), jnp.int32),
            pltpu.VMEM((R, D), jnp.float32),
            pltpu.VMEM((R, D), jnp.float32),
            pltpu.VMEM((NSEG, D), jnp.float32),
            pltpu.VMEM((L,), jnp.float32),
            pltpu.SemaphoreType.DMA,
            pltpu.SemaphoreType.DMA,
        ],
    )
    def k(feat_hbm, ids_hbm, psum_hbm, psq_hbm, ids_v, buf0_v, buf1_v,
          acc_v, sqout_v, sem0, sem1):
        wid = lax.axis_index("s") * NC + lax.axis_index("c")
        base = wid * ROWS_W
        bufs = [buf0_v, buf1_v]
        sems = [sem0, sem1]

        handles = {}
        for ch in range(min(2, NCH)):
            handles[ch] = pltpu.async_copy(
                feat_hbm.at[pl.ds(base + ch * R, R), :], bufs[ch % 2],
                sems[ch % 2])
        pltpu.sync_copy(ids_hbm.at[pl.ds(base, ROWS_W)], ids_v)

        zeros = jnp.zeros((L,), jnp.float32)

        def zacc(i, _):
            for c in range(CPD):
                acc_v[i, pl.ds(c * L, L)] = zeros
            return 0

        lax.fori_loop(0, NSEG, zacc, 0)

        iota = lax.iota(jnp.int32, L)
        sqs = tuple(zeros for _ in range(CPD))

        for ch in range(NCH):
            buf_v = bufs[ch % 2]
            handles[ch].wait()

            def grp(g, sqs):
                rb = g * L
                v = ids_v[pl.ds(ch * R + g * L, L)]
                lo = jnp.min(v)
                hi = jnp.max(v)

                def fast(sqs):
                    new = []
                    for c in range(CPD):
                        data = [buf_v[rb + r, pl.ds(c * L, L)]
                                for r in range(L)]
                        s = _tree_sum(data)
                        sq = _tree_sum([x * x for x in data])
                        plsc.addupdate(acc_v.at[lo, pl.ds(c * L, L)], s)
                        new.append(sqs[c] + sq)
                    return tuple(new)

                def slow(sqs):
                    sqs = list(sqs)
                    for r in range(L):
                        seg = jnp.sum(jnp.where(iota == r, v, 0))
                        for c in range(CPD):
                            data = buf_v[rb + r, pl.ds(c * L, L)]
                            plsc.addupdate(
                                acc_v.at[seg, pl.ds(c * L, L)], data)
                            sqs[c] = sqs[c] + data * data
                    return tuple(sqs)

                return lax.cond(lo == hi, fast, slow, sqs)

            sqs = lax.fori_loop(0, GPC, grp, sqs)
            if ch + 2 < NCH:
                handles[ch + 2] = pltpu.async_copy(
                    feat_hbm.at[pl.ds(base + (ch + 2) * R, R), :], buf_v,
                    sems[ch % 2])

        sqout_v[...] = _tree_sum(list(sqs))
        pltpu.sync_copy(acc_v, psum_hbm.at[wid])
        pltpu.sync_copy(sqout_v, psq_hbm.at[wid])

    return k(features, patient_ids)


def _epi_body(ps_ref, sq_ref, ids_ref, out_ref):
    sums = ps_ref[0]
    for t in range(1, NW):
        sums = sums + ps_ref[t]
    total_sq = jnp.sum(sq_ref[...])

    seg_iota = lax.broadcasted_iota(jnp.int32, (NSEG, D), 0)

    def cnt_body(r, cm):
        row = ids_ref[pl.ds(r, 1), :]
        m = jnp.broadcast_to(row, (NSEG, D)) == seg_iota
        return cm + m.astype(jnp.float32)

    cnt = lax.fori_loop(0, N // D, cnt_body,
                        jnp.zeros((NSEG, D), jnp.float32))
    cnt = jnp.broadcast_to(jnp.sum(cnt, axis=1, keepdims=True), (NSEG, D))

    safe = jnp.maximum(cnt, 1.0)
    cent = sums / safe
    csq = cent * cent
    within = total_sq - jnp.sum(cnt * csq)
    validf = (cnt > 0).astype(jnp.float32)
    kseg = jnp.sum(validf) / D
    csqsum = jnp.sum(validf * csq)
    svec = jnp.sum(validf * cent, axis=0, keepdims=True)
    ssq = jnp.sum(svec * svec)
    between = kseg * csqsum - ssq
    loss_pcsl = within / (between + EPS)
    loss_gpal = (csqsum - ssq / kseg) / kseg
    loss = LAMBDA_PCSL * loss_pcsl + LAMBDA_GPAL * loss_gpal
    out_ref[...] = jnp.broadcast_to(loss, (1, 1))


def kernel(features, patient_ids):
    psum, psq = _sc_partials(features, patient_ids)
    ids2d = patient_ids.reshape(N // D, D)
    out = pl.pallas_call(
        _epi_body,
        out_shape=jax.ShapeDtypeStruct((1, 1), jnp.float32),
    )(psum, psq, ids2d)
    return out[0, 0]
